# baseline (device time: 368225 ns/iter reference)
import jax
import jax.numpy as jnp
from jax import lax
from jax.experimental import pallas as pl
from jax.experimental.pallas import tpu as pltpu

N_DEV = 4
F32 = jnp.float32
BF16 = jnp.bfloat16
BK = 256


def _fused(A, B):
    M, K = A.shape
    _, N = B.shape
    mc = M // N_DEV
    nh = N // 2
    KS = K // BK

    def body(a_ref, b_ref, o_ref,
             acc_cw, acc_ccw, snd_cw, rcv_cw, snd_ccw, rcv_ccw,
             a_cw, a_ccw, bsl,
             ss_cw, rs_cw, ss_ccw, rs_ccw,
             credit_cw, credit_ccw, a_sems, b_sems, st_sems):
        d = lax.axis_index("i")
        left = lax.rem(d + N_DEV - 1, N_DEV)
        right = lax.rem(d + 1, N_DEV)

        barrier = pltpu.get_barrier_semaphore()
        for nbr in (left, right):
            pl.semaphore_signal(
                barrier, inc=1, device_id=(nbr,),
                device_id_type=pl.DeviceIdType.MESH,
            )
        pl.semaphore_wait(barrier, 2)

        def load_a(c, dst, i):
            cp = pltpu.make_async_copy(
                a_ref.at[pl.ds(c * mc, mc), :], dst, a_sems.at[i]
            )
            cp.start()
            return cp

        def load_b(kk):
            slot = kk % 2
            cp = pltpu.make_async_copy(
                b_ref.at[pl.ds(kk * BK, BK), :], bsl.at[slot], b_sems.at[slot]
            )
            cp.start()
            return cp

        def store(src, c, col0, sem):
            cp = pltpu.make_async_copy(
                src, o_ref.at[pl.ds(c * mc, mc), pl.ds(col0, nh)], sem
            )
            cp.start()
            return cp

        def rdma(src, dst, ssems, rsems, hop, dev):
            r = pltpu.make_async_remote_copy(
                src_ref=src, dst_ref=dst,
                send_sem=ssems.at[hop], recv_sem=rsems.at[hop],
                device_id=(dev,), device_id_type=pl.DeviceIdType.MESH,
            )
            r.start()
            return r

        def give_credit():
            pl.semaphore_signal(
                credit_cw, inc=1, device_id=(left,),
                device_id_type=pl.DeviceIdType.MESH,
            )
            pl.semaphore_signal(
                credit_ccw, inc=1, device_id=(right,),
                device_id_type=pl.DeviceIdType.MESH,
            )

        def take_credit():
            pl.semaphore_wait(credit_cw, 1)
            pl.semaphore_wait(credit_ccw, 1)

        def compute_chunks(acw, accw, first_b):
            pending = first_b
            for kk in range(KS):
                nxt = load_b(kk + 1) if kk < KS - 1 else None
                pending.wait()
                b = bsl.at[kk % 2]
                ksl = pl.ds(kk * BK, BK)
                p_cw = jnp.dot(
                    acw[:, ksl], b[:, :nh], preferred_element_type=F32
                )
                p_ccw = jnp.dot(
                    accw[:, ksl], b[:, nh:], preferred_element_type=F32
                )
                if kk == 0:
                    acc_cw[...] = p_cw
                    acc_ccw[...] = p_ccw
                else:
                    acc_cw[...] = acc_cw[...] + p_cw
                    acc_ccw[...] = acc_ccw[...] + p_ccw
                pending = nxt

        b0 = load_b(0)
        la = load_a(d, a_cw, 0)
        la.wait()
        compute_chunks(a_cw, a_cw, b0)

        la0 = load_a(lax.rem(d + N_DEV - 1, N_DEV), a_cw, 0)
        la1 = load_a(lax.rem(d + 1, N_DEV), a_ccw, 1)
        b0 = load_b(0)

        for s in range(3):
            snd_cw[...] = acc_cw[...].astype(BF16)
            snd_ccw[...] = acc_ccw[...].astype(BF16)
            if s > 0:
                take_credit()
            r_cw = rdma(snd_cw, rcv_cw, ss_cw, rs_cw, s, right)
            r_ccw = rdma(snd_ccw, rcv_ccw, ss_ccw, rs_ccw, s, left)
            la0.wait()
            la1.wait()
            compute_chunks(a_cw, a_ccw, b0)
            if s < 2:
                la0 = load_a(lax.rem(d - s - 2 + 2 * N_DEV, N_DEV), a_cw, 0)
                la1 = load_a(lax.rem(d + s + 2, N_DEV), a_ccw, 1)
                b0 = load_b(0)
            r_cw.wait()
            r_ccw.wait()
            acc_cw[...] = acc_cw[...] + rcv_cw[...].astype(F32)
            acc_ccw[...] = acc_ccw[...] + rcv_ccw[...].astype(F32)
            give_credit()

        own_cw = lax.rem(d + 1, N_DEV)
        own_ccw = lax.rem(d + 3, N_DEV)
        own0 = store(acc_cw, own_cw, 0, st_sems.at[0])
        own1 = store(acc_ccw, own_ccw, nh, st_sems.at[1])

        snd_cw[...] = acc_cw[...].astype(BF16)
        snd_ccw[...] = acc_ccw[...].astype(BF16)

        srcs = [(snd_cw, snd_ccw), (rcv_cw, rcv_ccw), (snd_cw, snd_ccw)]
        dsts = [(rcv_cw, rcv_ccw), (snd_cw, snd_ccw), (rcv_cw, rcv_ccw)]
        stgs = [(a_cw, a_ccw), (acc_cw, acc_ccw), (a_cw, a_ccw)]
        ag_stores = []
        for h in range(3):
            take_credit()
            r_cw = rdma(srcs[h][0], dsts[h][0], ss_cw, rs_cw, 3 + h, right)
            r_ccw = rdma(srcs[h][1], dsts[h][1], ss_ccw, rs_ccw, 3 + h, left)
            r_cw.wait()
            r_ccw.wait()
            if h < 2:
                give_credit()
            if h == 1:
                own0.wait()
                own1.wait()
            if h == 2:
                ag_stores[0][0].wait()
                ag_stores[0][1].wait()
            stg_cw, stg_ccw = stgs[h]
            stg_cw[...] = dsts[h][0][...].astype(F32)
            stg_ccw[...] = dsts[h][1][...].astype(F32)
            c_cw = lax.rem(d - h + 2 * N_DEV, N_DEV)
            c_ccw = lax.rem(d + h, N_DEV)
            s0 = store(stg_cw, c_cw, 0, st_sems.at[2 + 2 * h])
            s1 = store(stg_ccw, c_ccw, nh, st_sems.at[3 + 2 * h])
            ag_stores.append((s0, s1))

        for s0, s1 in ag_stores[1:]:
            s0.wait()
            s1.wait()

    return pl.pallas_call(
        body,
        in_specs=[
            pl.BlockSpec(memory_space=pl.ANY),
            pl.BlockSpec(memory_space=pl.ANY),
        ],
        out_specs=pl.BlockSpec(memory_space=pl.ANY),
        out_shape=jax.ShapeDtypeStruct((M, N), jnp.float32),
        scratch_shapes=[
            pltpu.VMEM((mc, nh), F32),
            pltpu.VMEM((mc, nh), F32),
            pltpu.VMEM((mc, nh), BF16),
            pltpu.VMEM((mc, nh), BF16),
            pltpu.VMEM((mc, nh), BF16),
            pltpu.VMEM((mc, nh), BF16),
            pltpu.VMEM((mc, K), F32),
            pltpu.VMEM((mc, K), F32),
            pltpu.VMEM((2, BK, N), F32),
            pltpu.SemaphoreType.DMA((6,)),
            pltpu.SemaphoreType.DMA((6,)),
            pltpu.SemaphoreType.DMA((6,)),
            pltpu.SemaphoreType.DMA((6,)),
            pltpu.SemaphoreType.REGULAR,
            pltpu.SemaphoreType.REGULAR,
            pltpu.SemaphoreType.DMA((2,)),
            pltpu.SemaphoreType.DMA((2,)),
            pltpu.SemaphoreType.DMA((8,)),
        ],
        compiler_params=pltpu.CompilerParams(
            collective_id=0,
            vmem_limit_bytes=60 * 1024 * 1024,
        ),
    )(A, B)


def kernel(A, B):
    return _fused(A, B)
